# baseline (speedup 1.0000x reference)
import functools

import jax
import jax.numpy as jnp
from jax import lax
from jax.experimental import pallas as pl
from jax.experimental.pallas import tpu as pltpu
from jax.experimental.pallas import tpu_sc as plsc

_GRID = 10
_NC, _NS = 2, 16
_NW = _NC * _NS


def _copy_x_body(xs_ref, xt_ref, u_ref, oxs_ref, oxt_ref, ou_ref):
    oxs_ref[...] = xs_ref[...]
    oxt_ref[...] = xt_ref[...]

    @pl.when(pl.program_id(0) == 0)
    def _():
        ou_ref[...] = u_ref[...]


def _sc_copy_body(rows_per_w, ea_hbm, out_hbm, buf):
    wid = lax.axis_index("s") * _NC + lax.axis_index("c")
    base = wid * rows_per_w
    pltpu.sync_copy(ea_hbm.at[pl.ds(base, rows_per_w)], buf)
    pltpu.sync_copy(buf, out_hbm.at[pl.ds(base, rows_per_w)])


def kernel(x_s, x_t, edge_index, edge_attr, u, batch_e, batch_s, batch_t):
    del edge_index, batch_e, batch_s, batch_t
    n_s, d_feat = x_s.shape
    e, d_edge = edge_attr.shape
    bx = n_s // _GRID

    xspecs = [
        pl.BlockSpec((bx, d_feat), lambda i: (i, 0)),
        pl.BlockSpec((bx, d_feat), lambda i: (i, 0)),
        pl.BlockSpec(u.shape, lambda i: (0, 0)),
    ]
    xs_o, xt_o, u_o = pl.pallas_call(
        _copy_x_body,
        grid=(_GRID,),
        in_specs=xspecs,
        out_specs=xspecs,
        out_shape=[
            jax.ShapeDtypeStruct(x_s.shape, x_s.dtype),
            jax.ShapeDtypeStruct(x_t.shape, x_t.dtype),
            jax.ShapeDtypeStruct(u.shape, u.dtype),
        ],
    )(x_s, x_t, u)

    rows_per_w = e // _NW
    mesh = plsc.VectorSubcoreMesh(core_axis_name="c", subcore_axis_name="s")
    sc_copy = pl.kernel(
        functools.partial(_sc_copy_body, rows_per_w),
        out_type=jax.ShapeDtypeStruct(edge_attr.shape, edge_attr.dtype),
        mesh=mesh,
        scratch_types=[pltpu.VMEM((rows_per_w, d_edge), edge_attr.dtype)],
        compiler_params=pltpu.CompilerParams(use_tc_tiling_on_sc=False),
    )
    ea_o = sc_copy(edge_attr)

    return (xs_o, xt_o, ea_o, u_o)
